# Initial kernel scaffold; baseline (speedup 1.0000x reference)
import jax
import jax.numpy as jnp
from jax.experimental import pallas as pl


def kernel(ids, table):
    raise NotImplementedError("write your pallas kernel here")


# speedup vs baseline: 14.0367x; 14.0367x over previous
import functools

import jax
import jax.numpy as jnp
from jax import lax
from jax.experimental import pallas as pl
from jax.experimental.pallas import tpu as pltpu
from jax.experimental.pallas import tpu_sc as plsc

_VOCAB = 100000
_DIM = 64
_B, _F = 4096, 26
_N = _B * _F
_NW = 32
_ROWS_PER_W = _N // _NW // 128
_K = 13
_HALF = _K * 128


def _gather_body(idx_hbm, table_hbm, out_hbm, idx_v, rows_v, sem):
    wid = lax.axis_index("s") * 2 + lax.axis_index("c")
    base = wid * (_ROWS_PER_W * 128)
    pltpu.sync_copy(idx_hbm.at[pl.ds(base, _ROWS_PER_W * 128)], idx_v)
    for h in range(2):
        copies = []
        for j in range(_K):
            copies.append(
                pltpu.async_copy(
                    table_hbm.at[idx_v.at[pl.ds((h * _K + j) * 128, 128)]],
                    rows_v.at[pl.ds(j * 128, 128)],
                    sem,
                )
            )
        for c in copies:
            c.wait()
        out0 = base + h * _HALF
        pltpu.sync_copy(rows_v, out_hbm.at[pl.ds(out0, _HALF)])


_gather = pl.kernel(
    _gather_body,
    mesh=plsc.VectorSubcoreMesh(core_axis_name="c", subcore_axis_name="s"),
    compiler_params=pltpu.CompilerParams(use_tc_tiling_on_sc=False),
    out_type=jax.ShapeDtypeStruct((_N, _DIM), jnp.float32),
    scratch_types=[
        pltpu.VMEM((_ROWS_PER_W * 128,), jnp.int32),
        pltpu.VMEM((_HALF, _DIM), jnp.float32),
        pltpu.SemaphoreType.DMA,
    ],
)


@jax.jit
def kernel(ids, table):
    idx = ids.reshape(_N)
    out = _gather(idx, table)
    return out.reshape(_B, _F, _DIM)


# speedup vs baseline: 14.0847x; 1.0034x over previous
import functools

import jax
import jax.numpy as jnp
from jax import lax
from jax.experimental import pallas as pl
from jax.experimental.pallas import tpu as pltpu
from jax.experimental.pallas import tpu_sc as plsc

_VOCAB = 100000
_DIM = 64
_B, _F = 4096, 26
_N = _B * _F
_NW = 32
_ROWS_PER_W = _N // _NW // 128
_K = 13
_HALF = _K * 128


def _gather_body(idx_hbm, table_hbm, out_hbm, idx_v, rows_v, sem):
    wid = lax.axis_index("s") * 2 + lax.axis_index("c")
    base = wid * (_ROWS_PER_W * 128)
    pltpu.sync_copy(idx_hbm.at[pl.ds(base, _ROWS_PER_W * 128)], idx_v)
    for h in range(2):
        pltpu.async_copy(
            table_hbm.at[idx_v.at[pl.ds(h * _HALF, _HALF)]],
            rows_v,
            sem,
        ).wait()
        out0 = base + h * _HALF
        pltpu.sync_copy(rows_v, out_hbm.at[pl.ds(out0, _HALF)])


_gather = pl.kernel(
    _gather_body,
    mesh=plsc.VectorSubcoreMesh(core_axis_name="c", subcore_axis_name="s"),
    compiler_params=pltpu.CompilerParams(use_tc_tiling_on_sc=False),
    out_type=jax.ShapeDtypeStruct((_N, _DIM), jnp.float32),
    scratch_types=[
        pltpu.VMEM((_ROWS_PER_W * 128,), jnp.int32),
        pltpu.VMEM((_HALF, _DIM), jnp.float32),
        pltpu.SemaphoreType.DMA,
    ],
)


@jax.jit
def kernel(ids, table):
    idx = ids.reshape(_N)
    out = _gather(idx, table)
    return out.reshape(_B, _F, _DIM)
